# baseline (speedup 1.0000x reference)
import functools

import jax
import jax.numpy as jnp
from jax import lax
from jax.experimental import pallas as pl
from jax.experimental.pallas import tpu as pltpu
from jax.experimental.pallas import tpu_sc as plsc

_B, _H, _W = 64, 512, 512
_NC, _NS, _L = 2, 16, 16
_NW = _NC * _NS
_BSC = 32
_SPT = _BSC // _NW
_CH = 64
_NCHUNK = _H // _CH
_JG = _W // _L


_GDN = lax.GatherDimensionNumbers(
    offset_dims=(), collapsed_slice_dims=(0,), start_index_map=(0,))


def _rot(x, step, lane):
    perm = (lane + step) & (_L - 1)
    return lax.gather(x, perm[:, None], dimension_numbers=_GDN,
                      slice_sizes=(1,),
                      mode=lax.GatherScatterMode.PROMISE_IN_BOUNDS)


def _xlane(x, op, lane):
    for step in (8, 4, 2, 1):
        x = op(x, _rot(x, step, lane))
    return x


def _sc_body(mask_hbm, wt_hbm, b_hbm, out_hbm, buf0, buf1, buf2, wbuf, bbuf,
             obuf, sem0, sem1, sem2):
    wid = lax.axis_index("s") * _NC + lax.axis_index("c")

    pltpu.sync_copy(wt_hbm, wbuf)
    pltpu.sync_copy(b_hbm, bbuf)

    lane = lax.iota(jnp.int32, _L)
    zero = lane * 0

    bufs = (buf0, buf1, buf2)
    sems = (sem0, sem1, sem2)
    nbuf = len(bufs)

    units = [(si, c) for si in range(_SPT) for c in range(_NCHUNK)]

    def start(u):
        si, c = units[u]
        s = wid * _SPT + si
        return pltpu.async_copy(
            mask_hbm.at[s, pl.ds(c * _CH, _CH)], bufs[u % nbuf],
            sems[u % nbuf])

    def row_body(r, carry, buf, c):
        cs = list(carry[:_JG])
        cnt0, cnt1, cnt2, cnt3, wsum, prev, minr, maxr = carry[_JG:]
        cnts = [cnt0, cnt1, cnt2, cnt3]
        for j in range(_JG):
            v = buf[r, pl.ds(_L * j, _L)]
            cs[j] = cs[j] + v
            cnts[j % 4] = cnts[j % 4] + v
        tot = (cnts[0] + cnts[1]) + (cnts[2] + cnts[3])
        diff = tot - prev
        lane_hit = diff != 0
        gr = zero + (c * _CH + r)
        minr = jnp.where(lane_hit, jnp.minimum(minr, gr), minr)
        maxr = jnp.where(lane_hit, jnp.maximum(maxr, gr), maxr)
        wsum = wsum + tot
        return tuple(cs) + (cnts[0], cnts[1], cnts[2], cnts[3], wsum, tot,
                            minr, maxr)

    copies = {0: start(0), 1: start(1)}
    for u in range(len(units)):
        si, c = units[u]
        if c == 0:
            acc = tuple(zero for _ in range(_JG)) + (
                zero, zero, zero, zero, zero, zero,
                zero + _H, zero - 1)
        if u + 2 < len(units):
            copies[u + 2] = start(u + 2)
        copies[u].wait()
        buf = bufs[u % nbuf]
        acc = lax.fori_loop(
            0, _CH, functools.partial(row_body, buf=buf, c=c), acc,
            unroll=False)
        if c == _NCHUNK - 1:
            cs = acc[:_JG]
            _, _, _, _, wsum, tot, minr, maxr = acc[_JG:]
            csum = zero
            minc = zero + _W
            maxc = zero - 1
            for j in range(_JG):
                cvec = lane + (_L * j)
                csum = csum + cs[j] * cvec
                colany = cs[j] != 0
                minc = jnp.where(colany, jnp.minimum(minc, cvec), minc)
                maxc = jnp.where(colany, jnp.maximum(maxc, cvec), maxc)
            count_v = _xlane(tot, jnp.add, lane)
            csum_v = _xlane(csum, jnp.add, lane)
            wsum_v = _xlane(wsum, jnp.add, lane)
            minr_v = _xlane(minr, jnp.minimum, lane)
            maxr_v = _xlane(maxr, jnp.maximum, lane)
            minc_v = _xlane(minc, jnp.minimum, lane)
            maxc_v = _xlane(maxc, jnp.maximum, lane)
            sumr_v = _H * count_v - wsum_v

            cntf = count_v.astype(jnp.float32)
            safe = jnp.maximum(cntf, 1.0)
            hf = jnp.float32(_H)
            wf = jnp.float32(_W)
            sumr_f = sumr_v.astype(jnp.float32)
            csum_f = csum_v.astype(jnp.float32)
            height = (maxr_v - minr_v).astype(jnp.float32)
            width = (maxc_v - minc_v).astype(jnp.float32)
            s0 = sumr_f / safe / hf
            s1 = csum_f / safe / wf
            s2 = height / hf
            s3 = width / wf
            s4 = cntf / (hf * wf)
            s5 = s2 * s3
            nz = cntf > 0.0
            zf = cntf * 0.0
            s0 = jnp.where(nz, s0, zf)
            s1 = jnp.where(nz, s1, zf)
            s2 = jnp.where(nz, s2, zf)
            s3 = jnp.where(nz, s3, zf)
            s4 = jnp.where(nz, s4, zf)
            s5 = jnp.where(nz, s5, zf)
            for i in range(128 // _L):
                sl = pl.ds(_L * i, _L)
                o = (bbuf[sl]
                     + s0 * wbuf[0, sl] + s1 * wbuf[1, sl]
                     + s2 * wbuf[2, sl] + s3 * wbuf[3, sl]
                     + s4 * wbuf[4, sl] + s5 * wbuf[5, sl])
                obuf[sl] = o
            s = wid * _SPT + si
            pltpu.sync_copy(obuf, out_hbm.at[s])


def _sc_part(mask3, wt, b):
    mesh = plsc.VectorSubcoreMesh(core_axis_name="c", subcore_axis_name="s")
    run = functools.partial(
        pl.kernel,
        mesh=mesh,
        out_type=jax.ShapeDtypeStruct((_BSC, 128), jnp.float32),
        scratch_types=[
            pltpu.VMEM((_CH, _W), jnp.int32),
            pltpu.VMEM((_CH, _W), jnp.int32),
            pltpu.VMEM((_CH, _W), jnp.int32),
            pltpu.VMEM((6, 128), jnp.float32),
            pltpu.VMEM((128,), jnp.float32),
            pltpu.VMEM((128,), jnp.float32),
            pltpu.SemaphoreType.DMA,
            pltpu.SemaphoreType.DMA,
            pltpu.SemaphoreType.DMA,
        ],
    )(_sc_body)
    return run(mask3, wt, b)


def _tc_body(mask_ref, w_ref, b_ref, out_ref):
    blk = mask_ref[0, 0]
    rowsum = jnp.sum(blk, axis=1, keepdims=True).astype(jnp.float32)
    colsum = jnp.sum(blk, axis=0, keepdims=True).astype(jnp.float32)
    r_iota = lax.broadcasted_iota(jnp.int32, (_H, 1), 0).astype(jnp.float32)
    c_iota = lax.broadcasted_iota(jnp.int32, (1, _W), 1).astype(jnp.float32)
    count = jnp.sum(rowsum)
    sum_r = jnp.sum(rowsum * r_iota)
    sum_c = jnp.sum(colsum * c_iota)
    neg = jnp.float32(-1e30)
    pos = jnp.float32(1e30)
    max_r = jnp.max(jnp.where(rowsum > 0, r_iota, neg))
    min_r = jnp.min(jnp.where(rowsum > 0, r_iota, pos))
    max_c = jnp.max(jnp.where(colsum > 0, c_iota, neg))
    min_c = jnp.min(jnp.where(colsum > 0, c_iota, pos))
    safe = jnp.maximum(count, 1.0)
    h = jnp.float32(_H)
    w = jnp.float32(_W)
    height = max_r - min_r
    width = max_c - min_c
    s0 = sum_r / safe / h
    s1 = sum_c / safe / w
    s2 = height / h
    s3 = width / w
    s4 = count / (h * w)
    s5 = height * width / (h * w)
    any_nz = count > 0.0
    zero = jnp.float32(0.0)
    s0 = jnp.where(any_nz, s0, zero)
    s1 = jnp.where(any_nz, s1, zero)
    s2 = jnp.where(any_nz, s2, zero)
    s3 = jnp.where(any_nz, s3, zero)
    s4 = jnp.where(any_nz, s4, zero)
    s5 = jnp.where(any_nz, s5, zero)
    wm = w_ref[...]
    out = (b_ref[0, :]
           + s0 * wm[:, 0] + s1 * wm[:, 1] + s2 * wm[:, 2]
           + s3 * wm[:, 3] + s4 * wm[:, 4] + s5 * wm[:, 5])
    out_ref[0, 0, :] = out


def _tc_part(mask, W, b2):
    n = _B - _BSC
    return pl.pallas_call(
        _tc_body,
        grid=(n,),
        in_specs=[
            pl.BlockSpec((1, 1, _H, _W), lambda i: (i + _BSC, 0, 0, 0)),
            pl.BlockSpec((128, 6), lambda i: (0, 0)),
            pl.BlockSpec((1, 128), lambda i: (0, 0)),
        ],
        out_specs=pl.BlockSpec((1, 1, 128), lambda i: (i, 0, 0)),
        out_shape=jax.ShapeDtypeStruct((n, 1, 128), jnp.float32),
    )(mask, W, b2).reshape(n, 128)


@jax.jit
def _hybrid(mask, W, b):
    mask3 = mask.reshape(_B, _H, _W)
    wt = W.T.reshape(6, 128)
    b2 = b.reshape(1, 128)
    out_sc = _sc_part(mask3, wt, b)
    out_tc = _tc_part(mask, W, b2)
    return jnp.concatenate([out_sc, out_tc], axis=0)


def kernel(mask, W, b):
    return _hybrid(mask, W, b)


# speedup vs baseline: 1.1900x; 1.1900x over previous
import functools

import jax
import jax.numpy as jnp
from jax import lax
from jax.experimental import pallas as pl
from jax.experimental.pallas import tpu as pltpu
from jax.experimental.pallas import tpu_sc as plsc

_B, _H, _W = 64, 512, 512
_NC, _NS, _L = 2, 16, 16
_NW = _NC * _NS
_SPT = _B // _NW
_CH = 64
_NCHUNK = _H // _CH
_JG = _W // _L


_GDN = lax.GatherDimensionNumbers(
    offset_dims=(), collapsed_slice_dims=(0,), start_index_map=(0,))


def _rot(x, step, lane):
    perm = (lane + step) & (_L - 1)
    return lax.gather(x, perm[:, None], dimension_numbers=_GDN,
                      slice_sizes=(1,),
                      mode=lax.GatherScatterMode.PROMISE_IN_BOUNDS)


def _xlane(x, op, lane):
    for step in (8, 4, 2, 1):
        x = op(x, _rot(x, step, lane))
    return x


def _sc_body(mask_hbm, wt_hbm, b_hbm, out_hbm, buf0, buf1, buf2, wbuf, bbuf,
             obuf0, obuf1, sem0, sem1, sem2, osem):
    wid = lax.axis_index("s") * _NC + lax.axis_index("c")

    lane = lax.iota(jnp.int32, _L)
    zero = lane * 0

    bufs = (buf0, buf1, buf2)
    sems = (sem0, sem1, sem2)
    obufs = (obuf0, obuf1)
    nbuf = len(bufs)

    units = [(si, c) for si in range(_SPT) for c in range(_NCHUNK)]

    def start(u):
        si, c = units[u]
        s = wid * _SPT + si
        return pltpu.async_copy(
            mask_hbm.at[s, pl.ds(c * _CH, _CH)], bufs[u % nbuf],
            sems[u % nbuf])

    def row_body(r, carry, buf, c):
        cs = list(carry[:_JG])
        cnt0, cnt1, cnt2, cnt3, wsum, prev, minr, maxr = carry[_JG:]
        cnts = [cnt0, cnt1, cnt2, cnt3]
        for j in range(_JG):
            v = buf[r, pl.ds(_L * j, _L)]
            cs[j] = cs[j] + v
            cnts[j % 4] = cnts[j % 4] + v
        tot = (cnts[0] + cnts[1]) + (cnts[2] + cnts[3])
        diff = tot - prev
        lane_hit = diff != 0
        gr = zero + (c * _CH + r)
        minr = jnp.where(lane_hit, jnp.minimum(minr, gr), minr)
        maxr = jnp.where(lane_hit, jnp.maximum(maxr, gr), maxr)
        wsum = wsum + tot
        return tuple(cs) + (cnts[0], cnts[1], cnts[2], cnts[3], wsum, tot,
                            minr, maxr)

    copies = {0: start(0), 1: start(1)}
    pltpu.sync_copy(wt_hbm, wbuf)
    pltpu.sync_copy(b_hbm, bbuf)
    out_copies = []
    for u in range(len(units)):
        si, c = units[u]
        if c == 0:
            acc = tuple(zero for _ in range(_JG)) + (
                zero, zero, zero, zero, zero, zero,
                zero + _H, zero - 1)
        if u + 2 < len(units):
            copies[u + 2] = start(u + 2)
        copies[u].wait()
        buf = bufs[u % nbuf]
        acc = lax.fori_loop(
            0, _CH, functools.partial(row_body, buf=buf, c=c), acc,
            unroll=False)
        if c == _NCHUNK - 1:
            cs = acc[:_JG]
            _, _, _, _, wsum, tot, minr, maxr = acc[_JG:]
            csum = zero
            minc = zero + _W
            maxc = zero - 1
            for j in range(_JG):
                cvec = lane + (_L * j)
                csum = csum + cs[j] * cvec
                colany = cs[j] != 0
                minc = jnp.where(colany, jnp.minimum(minc, cvec), minc)
                maxc = jnp.where(colany, jnp.maximum(maxc, cvec), maxc)
            count_v = _xlane(tot, jnp.add, lane)
            csum_v = _xlane(csum, jnp.add, lane)
            wsum_v = _xlane(wsum, jnp.add, lane)
            minr_v = _xlane(minr, jnp.minimum, lane)
            maxr_v = _xlane(maxr, jnp.maximum, lane)
            minc_v = _xlane(minc, jnp.minimum, lane)
            maxc_v = _xlane(maxc, jnp.maximum, lane)
            sumr_v = _H * count_v - wsum_v

            cntf = count_v.astype(jnp.float32)
            safe = jnp.maximum(cntf, 1.0)
            hf = jnp.float32(_H)
            wf = jnp.float32(_W)
            sumr_f = sumr_v.astype(jnp.float32)
            csum_f = csum_v.astype(jnp.float32)
            height = (maxr_v - minr_v).astype(jnp.float32)
            width = (maxc_v - minc_v).astype(jnp.float32)
            s0 = sumr_f / safe / hf
            s1 = csum_f / safe / wf
            s2 = height / hf
            s3 = width / wf
            s4 = cntf / (hf * wf)
            s5 = s2 * s3
            nz = cntf > 0.0
            zf = cntf * 0.0
            s0 = jnp.where(nz, s0, zf)
            s1 = jnp.where(nz, s1, zf)
            s2 = jnp.where(nz, s2, zf)
            s3 = jnp.where(nz, s3, zf)
            s4 = jnp.where(nz, s4, zf)
            s5 = jnp.where(nz, s5, zf)
            obuf = obufs[si]
            for i in range(128 // _L):
                sl = pl.ds(_L * i, _L)
                o = (bbuf[sl]
                     + s0 * wbuf[0, sl] + s1 * wbuf[1, sl]
                     + s2 * wbuf[2, sl] + s3 * wbuf[3, sl]
                     + s4 * wbuf[4, sl] + s5 * wbuf[5, sl])
                obuf[sl] = o
            s = wid * _SPT + si
            out_copies.append(pltpu.async_copy(obuf, out_hbm.at[s], osem))
    for oc in out_copies:
        oc.wait()


@jax.jit
def _sc_kernel(mask, W, b):
    mask3 = mask.reshape(_B, _H, _W)
    wt = W.T.reshape(6, 128)
    mesh = plsc.VectorSubcoreMesh(core_axis_name="c", subcore_axis_name="s")
    run = functools.partial(
        pl.kernel,
        mesh=mesh,
        out_type=jax.ShapeDtypeStruct((_B, 128), jnp.float32),
        scratch_types=[
            pltpu.VMEM((_CH, _W), jnp.int32),
            pltpu.VMEM((_CH, _W), jnp.int32),
            pltpu.VMEM((_CH, _W), jnp.int32),
            pltpu.VMEM((6, 128), jnp.float32),
            pltpu.VMEM((128,), jnp.float32),
            pltpu.VMEM((128,), jnp.float32),
            pltpu.VMEM((128,), jnp.float32),
            pltpu.SemaphoreType.DMA,
            pltpu.SemaphoreType.DMA,
            pltpu.SemaphoreType.DMA,
            pltpu.SemaphoreType.DMA,
        ],
    )(_sc_body)
    return run(mask3, wt, b)


def kernel(mask, W, b):
    return _sc_kernel(mask, W, b)
